# baseline (speedup 1.0000x reference)
import jax
import jax.numpy as jnp
import numpy as np
from jax.experimental import pallas as pl
from jax.experimental.pallas import tpu as pltpu

_B, _S, _D, _E = 1, 2048, 768, 8
_N = _B * _S
_MAX_THRESHOLD = 0.1
_GCOLS = 16


def _moe_body(x_ref, wg_ref, bias_ref, eb_ref, ew_ref, out_ref, w_scr):
    e = pl.program_id(0)

    @pl.when(e == 0)
    def _routing():
        logits = jnp.dot(x_ref[...], wg_ref[...],
                         preferred_element_type=jnp.float32) + bias_ref[...]
        g = logits[:, :_E]
        g = g - jnp.max(g, axis=-1, keepdims=True)
        g = jnp.exp(g)
        g = g / jnp.sum(g, axis=-1, keepdims=True)
        thr = jax.nn.sigmoid(logits[:, _E:_E + 1]) * _MAX_THRESHOLD
        ad = g - thr
        w = jnp.where(ad >= 0.0, ad, 0.0)
        s = jnp.sum(w, axis=-1, keepdims=True)
        w = w / jnp.where(s == 0.0, 1.0, s)
        w_scr[...] = w
        out_ref[...] = jnp.dot(w, eb_ref[...],
                               preferred_element_type=jnp.float32)

    lane = jax.lax.broadcasted_iota(jnp.int32, (_N, _E), 1)
    wcol = jnp.sum(jnp.where(lane == e, w_scr[...], 0.0),
                   axis=-1, keepdims=True)
    acc = jnp.dot(x_ref[...], ew_ref[0],
                  preferred_element_type=jnp.float32)
    out_ref[...] += wcol * acc


def kernel(inputs, gate_W, gate_b, thr_W, thr_b, exp_W, exp_b):
    flat = inputs.reshape(_N, _D)
    wg = jnp.zeros((_D, _GCOLS), dtype=jnp.float32)
    wg = wg.at[:, :_E].set(gate_W).at[:, _E:_E + 1].set(thr_W)
    bias = jnp.zeros((1, _GCOLS), dtype=jnp.float32)
    bias = bias.at[:, :_E].set(gate_b[None, :]).at[:, _E].set(thr_b[0])

    out = pl.pallas_call(
        _moe_body,
        grid=(_E,),
        in_specs=[
            pl.BlockSpec((_N, _D), lambda e: (0, 0)),
            pl.BlockSpec((_D, _GCOLS), lambda e: (0, 0)),
            pl.BlockSpec((1, _GCOLS), lambda e: (0, 0)),
            pl.BlockSpec((_E, _D), lambda e: (0, 0)),
            pl.BlockSpec((1, _D, _D), lambda e: (e, 0, 0)),
        ],
        out_specs=pl.BlockSpec((_N, _D), lambda e: (0, 0)),
        out_shape=jax.ShapeDtypeStruct((_N, _D), jnp.float32),
        scratch_shapes=[pltpu.VMEM((_N, _E), jnp.float32)],
        compiler_params=pltpu.CompilerParams(
            dimension_semantics=("arbitrary",),
        ),
    )(flat, wg, bias, exp_b, exp_W)
    return out.reshape(inputs.shape[:-1] + (_D,))
